# baseline (speedup 1.0000x reference)
import functools

import jax
import jax.numpy as jnp
from jax import lax
from jax.experimental import pallas as pl
from jax.experimental.pallas import tpu as pltpu
from jax.experimental.pallas import tpu_sc as plsc


def kernel(x, table):
    (B,) = x.shape
    V, D = table.shape

    info = plsc.get_sparse_core_info()
    NC, NS = info.num_cores, info.num_subcores
    NW = NC * NS

    CHUNK = 128
    n_chunks = B // CHUNK
    cpw = n_chunks // NW

    mesh = plsc.VectorSubcoreMesh(core_axis_name="c", subcore_axis_name="s")

    @functools.partial(
        pl.kernel,
        mesh=mesh,
        compiler_params=pltpu.CompilerParams(use_tc_tiling_on_sc=False),
        out_type=jax.ShapeDtypeStruct((n_chunks, CHUNK, D), jnp.float32),
        scratch_types=[
            pltpu.VMEM((cpw, CHUNK), jnp.int32),
            pltpu.VMEM((cpw, CHUNK, D), jnp.float32),
            pltpu.SemaphoreType.DMA,
        ],
    )
    def emb(idx_hbm, table_hbm, out_hbm, idx_v, rows_v, sem):
        wid = lax.axis_index("s") * NC + lax.axis_index("c")
        base = wid * cpw
        pltpu.sync_copy(idx_hbm.at[pl.ds(base, cpw)], idx_v)
        copies = [
            pltpu.async_copy(table_hbm.at[idx_v.at[j]], rows_v.at[j], sem)
            for j in range(cpw)
        ]
        for c in copies:
            c.wait()
        pltpu.sync_copy(rows_v, out_hbm.at[pl.ds(base, cpw)])

    x2d = x.astype(jnp.int32).reshape(n_chunks, CHUNK)
    out = emb(x2d, table)
    return out.reshape(B, D)


# speedup vs baseline: 1.6295x; 1.6295x over previous
import functools

import jax
import jax.numpy as jnp
from jax import lax
from jax.experimental import pallas as pl
from jax.experimental.pallas import tpu as pltpu
from jax.experimental.pallas import tpu_sc as plsc


def kernel(x, table):
    (B,) = x.shape
    V, D = table.shape

    info = plsc.get_sparse_core_info()
    NC, NS = info.num_cores, info.num_subcores
    NW = NC * NS

    NB = B // NW
    W = 64
    NWIN = NB // W

    mesh = plsc.VectorSubcoreMesh(core_axis_name="c", subcore_axis_name="s")

    @functools.partial(
        pl.kernel,
        mesh=mesh,
        out_type=jax.ShapeDtypeStruct((B, D), jnp.float32),
        scratch_types=[
            pltpu.VMEM((NB,), jnp.int32),
            pltpu.VMEM((NB, D), jnp.float32),
            pltpu.SemaphoreType.DMA,
        ],
    )
    def emb(idx_hbm, tbl_hbm, out_hbm, idx_v, rows_v, sem):
        wid = lax.axis_index("s") * NC + lax.axis_index("c")
        base = wid * NB
        pltpu.sync_copy(idx_hbm.at[pl.ds(base, NB)], idx_v)

        def window(w, _):
            def group(g, _):
                r = w * W + g * 16
                v = idx_v[pl.ds(r, 16)]
                for j in range(16):
                    pltpu.async_copy(
                        tbl_hbm.at[pl.ds(v[j], 1)],
                        rows_v.at[pl.ds(r + j, 1)],
                        sem,
                    )
                return ()

            lax.fori_loop(0, W // 16, group, ())
            pltpu.make_async_copy(
                tbl_hbm.at[pl.ds(0, W)], rows_v.at[pl.ds(w * W, W)], sem
            ).wait()
            return ()

        lax.fori_loop(0, NWIN, window, ())
        pltpu.sync_copy(rows_v, out_hbm.at[pl.ds(base, NB)])

    return emb(x.astype(jnp.int32), table)
